# baseline (speedup 1.0000x reference)
import functools

import jax
import jax.numpy as jnp
from jax import lax
from jax.experimental import pallas as pl
from jax.experimental.pallas import tpu as pltpu
from jax.experimental.pallas import tpu_sc as plsc

_CS = [64, 128, 256, 512]
_SHAPES = [(64, 70, 70), (128, 40, 40), (256, 24, 24), (512, 12, 12)]
_KS = [7, 5, 3, 1]
_TOPK = [4, 3, 3, 1]
_TNUM = 2
_H = 4
_NB = [616, 400, 576, 144]
_PREC = jax.lax.Precision.DEFAULT


def _ln(x, g, b, eps=1e-5):
    mu = jnp.mean(x, axis=-1, keepdims=True)
    var = jnp.mean((x - mu) ** 2, axis=-1, keepdims=True)
    return (x - mu) / jnp.sqrt(var + eps) * g + b


def _unfold(x, k):
    B, C, H, W = x.shape
    Hp, Wp = H // k, W // k
    u = x.reshape(B, C, Hp, k, Wp, k)
    return jnp.transpose(u, (0, 2, 4, 3, 5, 1)).reshape(B, Hp * Wp, k * k, C)


def _select_body(K, L, v, C, u_ref, o_ref):
    u = u_ref[0]
    ub = u.astype(jnp.bfloat16).astype(jnp.float32)
    s3 = jnp.sum(ub, axis=1, keepdims=True)
    rel = jnp.sum(ub * s3, axis=2, keepdims=True) * (1.0 / v)
    ii = jax.lax.broadcasted_iota(jnp.int32, (L, v, 1), 1)
    for kk in range(K):
        m = jnp.min(rel, axis=1, keepdims=True)
        ismin = rel == m
        idx = jnp.min(jnp.where(ismin, ii, v), axis=1, keepdims=True)
        mask = ii == idx
        o_ref[0, :, kk, :] = jnp.sum(u * mask.astype(jnp.float32), axis=1)
        rel = jnp.where(mask, jnp.inf, rel)


def _select(u, K):
    B, L, v, C = u.shape
    out = pl.pallas_call(
        functools.partial(_select_body, K, L, v, C),
        grid=(B,),
        in_specs=[pl.BlockSpec((1, L, v, C), lambda b: (b, 0, 0, 0))],
        out_specs=pl.BlockSpec((1, L, K, C), lambda b: (b, 0, 0, 0)),
        out_shape=jax.ShapeDtypeStruct((B, L, K, C), jnp.float32),
    )(u)
    return out.reshape(B, L * K, C)


def _rel_body(L, v, vp, u_ref, o_ref):
    u = u_ref[0]
    ub = u.astype(jnp.bfloat16).astype(jnp.float32)
    s3 = jnp.sum(ub, axis=1, keepdims=True)
    rel = jnp.sum(ub * s3, axis=2, keepdims=True) * (1.0 / v)
    o_ref[0, :, 0:v, :] = rel
    if vp > v:
        o_ref[0, :, v:vp, :] = jnp.full((L, vp - v, 1), jnp.inf, jnp.float32)


def _rel(u, vp):
    B, L, v, C = u.shape
    out = pl.pallas_call(
        functools.partial(_rel_body, L, v, vp),
        grid=(B,),
        in_specs=[pl.BlockSpec((1, L, v, C), lambda b: (b, 0, 0, 0))],
        out_specs=pl.BlockSpec((1, L, vp, 1), lambda b: (b, 0, 0, 0)),
        out_shape=jax.ShapeDtypeStruct((B, L, vp, 1), jnp.float32),
    )(u)
    return out.reshape(B * L, vp)


_NC = 2
_NS = 16
_NW = _NC * _NS
_PPW = 8
_IOTA16 = None


def _lane_perm(x, perm):
    dn = lax.GatherDimensionNumbers(offset_dims=(), collapsed_slice_dims=(0,),
                                    start_index_map=(0,))
    return lax.gather(x, perm[:, None], dn, (1,),
                      mode=lax.GatherScatterMode.PROMISE_IN_BOUNDS)


def _lane_min_bcast(x, ii):
    for sh in (8, 4, 2, 1):
        x = jnp.minimum(x, _lane_perm(x, jnp.bitwise_and(ii + sh, 15)))
    return x

def _sc_select_body(cfg, *refs):
    nlvl = len(cfg)
    rel_refs = refs[0:nlvl]
    u_refs = refs[nlvl:2 * nlvl]
    o_refs = refs[2 * nlvl:3 * nlvl]
    scr = refs[3 * nlvl:]
    sem = scr[-1]
    wid = lax.axis_index("s") * _NC + lax.axis_index("c")
    ii = lax.broadcasted_iota(jnp.int32, (16,), 0)
    big = jnp.int32(1 << 20)
    inf = jnp.float32(jnp.inf)
    for lvl in range(nlvl):
        P, v, vp, K, C, _Creal = cfg[lvl]
        rv = scr[3 * lvl + 0]
        ib = scr[3 * lvl + 1]
        rw = scr[3 * lvl + 2]
        base = wid * _PPW
        pltpu.sync_copy(rel_refs[lvl].at[pl.ds(base * vp, _PPW * vp)], rv)
        nj = vp // 16
        nib = -(-(_PPW * K) // 16)
        basev = jnp.full((16,), wid, jnp.int32) * _PPW
        acc = [jnp.zeros((16,), jnp.int32) for _ in range(nib)]
        for p in range(_PPW):
            cur = [rv[pl.ds(p * vp + j * 16, 16)] for j in range(nj)]
            for kk in range(K):
                m = _lane_min_bcast(functools.reduce(jnp.minimum, cur), ii)
                cand = [jnp.where(cur[j] == m, ii + j * 16, big)
                        for j in range(nj)]
                idxv = _lane_min_bcast(functools.reduce(jnp.minimum, cand), ii)
                for j in range(nj):
                    cur[j] = jnp.where(ii + j * 16 == idxv, inf, cur[j])
                g = jnp.minimum((basev + p) * v + idxv, P * v - 1)
                pos = p * K + kk
                acc[pos // 16] = jnp.where(ii == (pos % 16), g, acc[pos // 16])
        for j in range(nib):
            ib[pl.ds(j * 16, 16)] = acc[j]
        pltpu.async_copy(u_refs[lvl].at[ib], rw, sem).wait()
        pltpu.sync_copy(rw.at[pl.ds(0, _PPW * K)],
                        o_refs[lvl].at[pl.ds(base * K, _PPW * K)])


def _sc_select(rels, us, cfg):
    mesh = plsc.VectorSubcoreMesh(core_axis_name="c", subcore_axis_name="s")
    out_type = [jax.ShapeDtypeStruct((_NW * _PPW * K, C), jnp.float32)
                for (P, v, vp, K, C, _Cr) in cfg]
    scratch = []
    for (P, v, vp, K, C, _Cr) in cfg:
        nib = -(-(_PPW * K) // 16) * 16
        scratch += [pltpu.VMEM((_PPW * vp,), jnp.float32),
                    pltpu.VMEM((nib,), jnp.int32),
                    pltpu.VMEM((nib, C), jnp.float32)]
    scratch.append(pltpu.SemaphoreType.DMA)
    fn = pl.kernel(functools.partial(_sc_select_body, cfg),
                   out_type=out_type, mesh=mesh, scratch_types=scratch)
    return fn(*rels, *us)


def _attn_body(Ci, Ms, Nb, q_ref,
               s0, s1, s2, qg, qb, kg0, kb0, kg1, kb1, kg2, kb2,
               wq, wk0, wv0, wk1, wv1, wk2, wv2, wout,
               o_ref, k_scr, v_scr, os_scr):
    dh = Ci
    sk = max(dh, 128)
    s_refs = (s0, s1, s2)
    kg = (kg0, kg1, kg2)
    kb = (kb0, kb1, kb2)
    wk = (wk0, wk1, wk2)
    wv = (wv0, wv1, wv2)
    offs = (0, Ms[0], Ms[0] + Ms[1])

    @pl.when(pl.program_id(1) == 0)
    def _():
        for c in range(3):
            ln = _ln(s_refs[c][0], kg[c][0], kb[c][0])
            k_scr[offs[c]:offs[c] + Ms[c], :] = jax.lax.dot(
                ln, wk[c][...], precision=_PREC).astype(jnp.bfloat16)
            v_scr[offs[c]:offs[c] + Ms[c], :] = jax.lax.dot(ln, wv[c][...], precision=_PREC)

    x = _ln(q_ref[0], qg[0], qb[0])
    q_all = jax.lax.dot(x, wq[...], precision=_PREC)
    q_b = q_all.astype(jnp.bfloat16)
    for h in range(_H):
        qh = q_b[:, h * sk:(h + 1) * sk]
        for c in range(3):
            kc = k_scr[offs[c]:offs[c] + Ms[c], h * sk:(h + 1) * sk]
            dots = jax.lax.dot_general(qh, kc, (((1,), (1,)), ((), ())),
                                       preferred_element_type=jnp.float32)
            e = jnp.exp(dots)
            r = 1.0 / jnp.sum(e, axis=1, keepdims=True)
            vc = v_scr[offs[c]:offs[c] + Ms[c], h * sk:(h + 1) * sk]
            av = jax.lax.dot(e, vc, precision=_PREC)
            os_scr[:, (h * 3 + c) * sk:(h * 3 + c + 1) * sk] = av * r
    o_ref[0] = jax.lax.dot(os_scr[...], wout[...], precision=_PREC)


def _pad_heads(w, dh, sk):
    if sk == dh:
        return w
    rows = w.shape[0]
    w = w.reshape(rows, _H, dh)
    w = jnp.pad(w, ((0, 0), (0, 0), (0, sk - dh)))
    return w.reshape(rows, _H * sk)


def _attn(query, skips, ap, i):
    B, N, Ci = query.shape
    dh = Ci
    sk = max(dh, 128)
    inner = _H * Ci
    Ms = tuple(s.shape[1] for s in skips)
    ds = tuple(s.shape[2] for s in skips)
    Mtot = sum(Ms)
    Nb = _NB[i]
    nblk = pl.cdiv(N, Nb)
    scale = dh ** -0.5

    wq = _pad_heads(ap['Wq'].T * scale, dh, sk)
    wks, wvs = [], []
    for c in range(3):
        wkvT = ap['Wkv'][c].T
        wks.append(_pad_heads(wkvT[:, :inner], dh, sk))
        wvs.append(_pad_heads(wkvT[:, inner:], dh, sk))
    woutT = ap['Wout'].T
    if sk != dh:
        w3 = woutT.reshape(3 * _H, dh, Ci)
        w3 = jnp.pad(w3, ((0, 0), (0, sk - dh), (0, 0)))
        woutT = w3.reshape(3 * _H * sk, Ci)

    def full(a):
        nd = a.ndim
        return pl.BlockSpec(a.shape, lambda b, n: (0,) * nd)

    qg = ap['qn_g'].reshape(1, Ci)
    qb = ap['qn_b'].reshape(1, Ci)
    kgs = [g.reshape(1, -1) for g in ap['kvn_g']]
    kbs = [b.reshape(1, -1) for b in ap['kvn_b']]

    operands = [query] + list(skips) + [qg, qb,
                kgs[0], kbs[0], kgs[1], kbs[1], kgs[2], kbs[2],
                wq, wks[0], wvs[0], wks[1], wvs[1], wks[2], wvs[2], woutT]
    in_specs = [pl.BlockSpec((1, Nb, Ci), lambda b, n: (b, n, 0))]
    for c in range(3):
        in_specs.append(pl.BlockSpec((1, Ms[c], ds[c]),
                                     lambda b, n: (b, 0, 0)))
    for a in operands[4:]:
        in_specs.append(full(a))

    return pl.pallas_call(
        functools.partial(_attn_body, Ci, Ms, Nb),
        grid=(B, nblk),
        in_specs=in_specs,
        out_specs=pl.BlockSpec((1, Nb, Ci), lambda b, n: (b, n, 0)),
        out_shape=jax.ShapeDtypeStruct((B, N, Ci), jnp.float32),
        scratch_shapes=[
            pltpu.VMEM((Mtot, _H * sk), jnp.bfloat16),
            pltpu.VMEM((Mtot, _H * sk), jnp.float32),
            pltpu.VMEM((Nb, 3 * _H * sk), jnp.float32),
        ],
    )(*operands)


def _smla(xs, blocks):
    B = xs[0].shape[0]
    tmp_q, tmp_sk = [], []
    rels, uflats, cfg = [], [], []
    for i, x in enumerate(xs):
        C, Hh, Ww = _SHAPES[i]
        tmp_q.append(x.reshape(B, C, Hh * Ww).transpose(0, 2, 1))
        u = _unfold(x, _KS[i])
        if _KS[i] == 1:
            tmp_sk.append(u.reshape(B, -1, C))
            continue
        L, v = u.shape[1], u.shape[2]
        vp = max(16, -(-v // 16) * 16)
        P = B * L
        ppad = _NW * _PPW
        r = _rel(u, vp)
        r = jnp.pad(r, ((0, ppad - P), (0, 0)), constant_values=jnp.inf)
        rels.append(r.reshape(-1))
        uf = u.reshape(P * v, C)
        if C < 128:
            uf = jnp.pad(uf, ((0, 0), (0, 128 - C)))
        uflats.append(uf)
        cfg.append((P, v, vp, _TOPK[i], max(C, 128), C))
        tmp_sk.append(i)
    sels = _sc_select(rels, uflats, tuple(cfg))
    si = 0
    for i in range(4):
        if isinstance(tmp_sk[i], int):
            P, v, vp, K, Cp, C = cfg[si]
            L = P // B
            tmp_sk[i] = sels[si][:P * K, :C].reshape(B, L * K, C)
            si += 1
    new = []
    for idx in range(4):
        new.append(_attn(tmp_q[idx],
                         [tmp_sk[j] for j in range(4) if j != idx],
                         blocks[idx], idx))
    outs = []
    for i, ns in enumerate(new):
        C, Hh, Ww = _SHAPES[i]
        outs.append(ns.transpose(0, 2, 1).reshape(B, C, Hh, Ww))
    return outs


def kernel(x0, x1, x2, x3, params):
    xs = [x0, x1, x2, x3]
    for t in range(_TNUM):
        xs = _smla(xs, params[t])
    return tuple(xs)


# speedup vs baseline: 1.1028x; 1.1028x over previous
import functools

import jax
import jax.numpy as jnp
from jax import lax
from jax.experimental import pallas as pl
from jax.experimental.pallas import tpu as pltpu
from jax.experimental.pallas import tpu_sc as plsc

_CS = [64, 128, 256, 512]
_SHAPES = [(64, 70, 70), (128, 40, 40), (256, 24, 24), (512, 12, 12)]
_KS = [7, 5, 3, 1]
_TOPK = [4, 3, 3, 1]
_TNUM = 2
_H = 4
_NB = [616, 400, 576, 144]
_PREC = jax.lax.Precision.DEFAULT


def _ln(x, g, b, eps=1e-5):
    mu = jnp.mean(x, axis=-1, keepdims=True)
    var = jnp.mean((x - mu) ** 2, axis=-1, keepdims=True)
    return (x - mu) / jnp.sqrt(var + eps) * g + b


def _unfold(x, k):
    B, C, H, W = x.shape
    Hp, Wp = H // k, W // k
    u = x.reshape(B, C, Hp, k, Wp, k)
    return jnp.transpose(u, (0, 2, 4, 3, 5, 1)).reshape(B, Hp * Wp, k * k, C)


def _select_body(K, L, v, C, u_ref, o_ref):
    u = u_ref[0]
    ub = u.astype(jnp.bfloat16).astype(jnp.float32)
    s3 = jnp.sum(ub, axis=1, keepdims=True)
    rel = jnp.sum(ub * s3, axis=2, keepdims=True) * (1.0 / v)
    ii = jax.lax.broadcasted_iota(jnp.int32, (L, v, 1), 1)
    for kk in range(K):
        m = jnp.min(rel, axis=1, keepdims=True)
        ismin = rel == m
        idx = jnp.min(jnp.where(ismin, ii, v), axis=1, keepdims=True)
        mask = ii == idx
        o_ref[0, :, kk, :] = jnp.sum(u * mask.astype(jnp.float32), axis=1)
        rel = jnp.where(mask, jnp.inf, rel)


def _select(u, K):
    B, L, v, C = u.shape
    out = pl.pallas_call(
        functools.partial(_select_body, K, L, v, C),
        grid=(B,),
        in_specs=[pl.BlockSpec((1, L, v, C), lambda b: (b, 0, 0, 0))],
        out_specs=pl.BlockSpec((1, L, K, C), lambda b: (b, 0, 0, 0)),
        out_shape=jax.ShapeDtypeStruct((B, L, K, C), jnp.float32),
    )(u)
    return out.reshape(B, L * K, C)


def _rel3_body(shapes, u0, u1, u2, o0, o1, o2):
    for u_ref, o_ref, (L, v, vp, C) in zip((u0, u1, u2), (o0, o1, o2), shapes):
        u = u_ref[0]
        ub = u.astype(jnp.bfloat16).astype(jnp.float32)
        s3 = jnp.sum(ub, axis=1, keepdims=True)
        rel = jnp.sum(ub * s3, axis=2, keepdims=True) * (1.0 / v)
        o_ref[0, :, 0:v, :] = rel
        if vp > v:
            o_ref[0, :, v:vp, :] = jnp.full((L, vp - v, 1), jnp.inf, jnp.float32)


def _rel3(us, vps):
    B = us[0].shape[0]
    shapes = tuple((u.shape[1], u.shape[2], vp, u.shape[3])
                   for u, vp in zip(us, vps))
    outs = pl.pallas_call(
        functools.partial(_rel3_body, shapes),
        grid=(B,),
        in_specs=[pl.BlockSpec((1, L, v, C), lambda b: (b, 0, 0, 0))
                  for (L, v, vp, C) in shapes],
        out_specs=[pl.BlockSpec((1, L, vp, 1), lambda b: (b, 0, 0, 0))
                   for (L, v, vp, C) in shapes],
        out_shape=[jax.ShapeDtypeStruct((B, L, vp, 1), jnp.float32)
                   for (L, v, vp, C) in shapes],
    )(*us)
    return [o.reshape(-1, vp) for o, (L, v, vp, C) in zip(outs, shapes)]


_NC = 2
_NS = 16
_NW = _NC * _NS
_PPW = 8
_IOTA16 = None


def _lane_perm(x, perm):
    dn = lax.GatherDimensionNumbers(offset_dims=(), collapsed_slice_dims=(0,),
                                    start_index_map=(0,))
    return lax.gather(x, perm[:, None], dn, (1,),
                      mode=lax.GatherScatterMode.PROMISE_IN_BOUNDS)


def _lane_min_bcast(x, ii):
    for sh in (8, 4, 2, 1):
        x = jnp.minimum(x, _lane_perm(x, jnp.bitwise_and(ii + sh, 15)))
    return x

def _sc_select_body(cfg, *refs):
    nlvl = len(cfg)
    rel_refs = refs[0:nlvl]
    u_refs = refs[nlvl:2 * nlvl]
    o_refs = refs[2 * nlvl:3 * nlvl]
    scr = refs[3 * nlvl:]
    sem = scr[-1]
    wid = lax.axis_index("s") * _NC + lax.axis_index("c")
    ii = lax.broadcasted_iota(jnp.int32, (16,), 0)
    big = jnp.int32(1 << 20)
    inf = jnp.float32(jnp.inf)
    base = wid * _PPW
    basev = jnp.full((16,), wid, jnp.int32) * _PPW
    rsems = scr[-2 * nlvl - 1:-nlvl - 1]
    gsems = scr[-nlvl - 1:-1]
    rcps = []
    for lvl in range(nlvl):
        P, v, vp, K, C, _Creal = cfg[lvl]
        rv = scr[3 * lvl + 0]
        rcps.append(pltpu.async_copy(
            rel_refs[lvl].at[pl.ds(base * vp, _PPW * vp)], rv, rsems[lvl]))
    gcps = []
    for lvl in range(nlvl):
        P, v, vp, K, C, _Creal = cfg[lvl]
        rv = scr[3 * lvl + 0]
        ib = scr[3 * lvl + 1]
        rw = scr[3 * lvl + 2]
        rcps[lvl].wait()
        nj = vp // 16
        nib = -(-(_PPW * K) // 16)
        acc = [jnp.zeros((16,), jnp.int32) for _ in range(nib)]
        for p in range(_PPW):
            cur = [rv[pl.ds(p * vp + j * 16, 16)] for j in range(nj)]
            for kk in range(K):
                m = _lane_min_bcast(functools.reduce(jnp.minimum, cur), ii)
                cand = [jnp.where(cur[j] == m, ii + j * 16, big)
                        for j in range(nj)]
                idxv = _lane_min_bcast(functools.reduce(jnp.minimum, cand), ii)
                for j in range(nj):
                    cur[j] = jnp.where(ii + j * 16 == idxv, inf, cur[j])
                g = jnp.minimum((basev + p) * v + idxv, P * v - 1)
                pos = p * K + kk
                acc[pos // 16] = jnp.where(ii == (pos % 16), g, acc[pos // 16])
        for j in range(nib):
            ib[pl.ds(j * 16, 16)] = acc[j]
        gcps.append(pltpu.async_copy(u_refs[lvl].at[ib], rw, gsems[lvl]))
    for lvl in range(nlvl):
        P, v, vp, K, C, _Creal = cfg[lvl]
        rw = scr[3 * lvl + 2]
        gcps[lvl].wait()
        pltpu.sync_copy(rw.at[pl.ds(0, _PPW * K)],
                        o_refs[lvl].at[pl.ds(base * K, _PPW * K)])


def _sc_select(rels, us, cfg):
    mesh = plsc.VectorSubcoreMesh(core_axis_name="c", subcore_axis_name="s")
    out_type = [jax.ShapeDtypeStruct((_NW * _PPW * K, C), jnp.float32)
                for (P, v, vp, K, C, _Cr) in cfg]
    scratch = []
    for (P, v, vp, K, C, _Cr) in cfg:
        nib = -(-(_PPW * K) // 16) * 16
        scratch += [pltpu.VMEM((_PPW * vp,), jnp.float32),
                    pltpu.VMEM((nib,), jnp.int32),
                    pltpu.VMEM((nib, C), jnp.float32)]
    for _ in range(2 * len(cfg)):
        scratch.append(pltpu.SemaphoreType.DMA)
    scratch.append(pltpu.SemaphoreType.DMA)
    fn = pl.kernel(functools.partial(_sc_select_body, cfg),
                   out_type=out_type, mesh=mesh, scratch_types=scratch)
    return fn(*rels, *us)


def _attn_body(Ci, Ms, Nb, q_ref,
               s0, s1, s2, qg, qb, kg0, kb0, kg1, kb1, kg2, kb2,
               wq, wk0, wv0, wk1, wv1, wk2, wv2, wout,
               o_ref, k_scr, v_scr, os_scr):
    dh = Ci
    sk = max(dh, 128)
    s_refs = (s0, s1, s2)
    kg = (kg0, kg1, kg2)
    kb = (kb0, kb1, kb2)
    wk = (wk0, wk1, wk2)
    wv = (wv0, wv1, wv2)
    offs = (0, Ms[0], Ms[0] + Ms[1])

    @pl.when(pl.program_id(1) == 0)
    def _():
        for c in range(3):
            ln = _ln(s_refs[c][0], kg[c][0], kb[c][0])
            k_scr[offs[c]:offs[c] + Ms[c], :] = jax.lax.dot(ln, wk[c][...], precision=_PREC)
            v_scr[offs[c]:offs[c] + Ms[c], :] = jax.lax.dot(ln, wv[c][...], precision=_PREC)

    x = _ln(q_ref[0], qg[0], qb[0])
    q_all = jax.lax.dot(x, wq[...], precision=_PREC)
    for h in range(_H):
        qh = q_all[:, h * sk:(h + 1) * sk]
        for c in range(3):
            kc = k_scr[offs[c]:offs[c] + Ms[c], h * sk:(h + 1) * sk]
            dots = jax.lax.dot_general(qh, kc, (((1,), (1,)), ((), ())), precision=_PREC)
            e = jnp.exp(dots)
            r = 1.0 / jnp.sum(e, axis=1, keepdims=True)
            vc = v_scr[offs[c]:offs[c] + Ms[c], h * sk:(h + 1) * sk]
            av = jax.lax.dot(e, vc, precision=_PREC)
            os_scr[:, (h * 3 + c) * sk:(h * 3 + c + 1) * sk] = av * r
    o_ref[0] = jax.lax.dot(os_scr[...], wout[...], precision=_PREC)


def _pad_heads(w, dh, sk):
    if sk == dh:
        return w
    rows = w.shape[0]
    w = w.reshape(rows, _H, dh)
    w = jnp.pad(w, ((0, 0), (0, 0), (0, sk - dh)))
    return w.reshape(rows, _H * sk)


def _attn(query, skips, ap, i):
    B, N, Ci = query.shape
    dh = Ci
    sk = max(dh, 128)
    inner = _H * Ci
    Ms = tuple(s.shape[1] for s in skips)
    ds = tuple(s.shape[2] for s in skips)
    Mtot = sum(Ms)
    Nb = _NB[i]
    nblk = pl.cdiv(N, Nb)
    scale = dh ** -0.5

    wq = _pad_heads(ap['Wq'].T * scale, dh, sk)
    wks, wvs = [], []
    for c in range(3):
        wkvT = ap['Wkv'][c].T
        wks.append(_pad_heads(wkvT[:, :inner], dh, sk))
        wvs.append(_pad_heads(wkvT[:, inner:], dh, sk))
    woutT = ap['Wout'].T
    if sk != dh:
        w3 = woutT.reshape(3 * _H, dh, Ci)
        w3 = jnp.pad(w3, ((0, 0), (0, sk - dh), (0, 0)))
        woutT = w3.reshape(3 * _H * sk, Ci)

    def full(a):
        nd = a.ndim
        return pl.BlockSpec(a.shape, lambda b, n: (0,) * nd)

    qg = ap['qn_g'].reshape(1, Ci)
    qb = ap['qn_b'].reshape(1, Ci)
    kgs = [g.reshape(1, -1) for g in ap['kvn_g']]
    kbs = [b.reshape(1, -1) for b in ap['kvn_b']]

    operands = [query] + list(skips) + [qg, qb,
                kgs[0], kbs[0], kgs[1], kbs[1], kgs[2], kbs[2],
                wq, wks[0], wvs[0], wks[1], wvs[1], wks[2], wvs[2], woutT]
    in_specs = [pl.BlockSpec((1, Nb, Ci), lambda b, n: (b, n, 0))]
    for c in range(3):
        in_specs.append(pl.BlockSpec((1, Ms[c], ds[c]),
                                     lambda b, n: (b, 0, 0)))
    for a in operands[4:]:
        in_specs.append(full(a))

    return pl.pallas_call(
        functools.partial(_attn_body, Ci, Ms, Nb),
        grid=(B, nblk),
        in_specs=in_specs,
        out_specs=pl.BlockSpec((1, Nb, Ci), lambda b, n: (b, n, 0)),
        out_shape=jax.ShapeDtypeStruct((B, N, Ci), jnp.float32),
        scratch_shapes=[
            pltpu.VMEM((Mtot, _H * sk), jnp.float32),
            pltpu.VMEM((Mtot, _H * sk), jnp.float32),
            pltpu.VMEM((Nb, 3 * _H * sk), jnp.float32),
        ],
    )(*operands)


def _smla(xs, blocks):
    B = xs[0].shape[0]
    tmp_q, tmp_sk = [], []
    rels, uflats, cfg, us, vps = [], [], [], [], []
    for i, x in enumerate(xs):
        C, Hh, Ww = _SHAPES[i]
        tmp_q.append(x.reshape(B, C, Hh * Ww).transpose(0, 2, 1))
        u = _unfold(x, _KS[i])
        if _KS[i] == 1:
            tmp_sk.append(u.reshape(B, -1, C))
            continue
        L, v = u.shape[1], u.shape[2]
        vp = max(16, -(-v // 16) * 16)
        P = B * L
        ppad = _NW * _PPW
        us.append(u)
        vps.append(vp)
        uf = u.reshape(P * v, C)
        if C < 128:
            uf = jnp.pad(uf, ((0, 0), (0, 128 - C)))
        uflats.append(uf)
        cfg.append((P, v, vp, _TOPK[i], max(C, 128), C))
        tmp_sk.append(i)
    ppad = _NW * _PPW
    for r, (P, v, vp, K, Cp, C) in zip(_rel3(us, vps), cfg):
        r = jnp.pad(r, ((0, ppad - P), (0, 0)), constant_values=jnp.inf)
        rels.append(r.reshape(-1))
    sels = _sc_select(rels, uflats, tuple(cfg))
    si = 0
    for i in range(4):
        if isinstance(tmp_sk[i], int):
            P, v, vp, K, Cp, C = cfg[si]
            L = P // B
            tmp_sk[i] = sels[si][:P * K, :C].reshape(B, L * K, C)
            si += 1
    new = []
    for idx in range(4):
        new.append(_attn(tmp_q[idx],
                         [tmp_sk[j] for j in range(4) if j != idx],
                         blocks[idx], idx))
    outs = []
    for i, ns in enumerate(new):
        C, Hh, Ww = _SHAPES[i]
        outs.append(ns.transpose(0, 2, 1).reshape(B, C, Hh, Ww))
    return outs


def kernel(x0, x1, x2, x3, params):
    xs = [x0, x1, x2, x3]
    for t in range(_TNUM):
        xs = _smla(xs, params[t])
    return tuple(xs)


# speedup vs baseline: 1.1096x; 1.0062x over previous
import functools

import jax
import jax.numpy as jnp
from jax import lax
from jax.experimental import pallas as pl
from jax.experimental.pallas import tpu as pltpu
from jax.experimental.pallas import tpu_sc as plsc

_CS = [64, 128, 256, 512]
_SHAPES = [(64, 70, 70), (128, 40, 40), (256, 24, 24), (512, 12, 12)]
_KS = [7, 5, 3, 1]
_TOPK = [4, 3, 3, 1]
_TNUM = 2
_H = 4
_NB = [640, 1600, 576, 144]
_PREC = jax.lax.Precision.DEFAULT


def _ln(x, g, b, eps=1e-5):
    mu = jnp.mean(x, axis=-1, keepdims=True)
    var = jnp.mean((x - mu) ** 2, axis=-1, keepdims=True)
    return (x - mu) / jnp.sqrt(var + eps) * g + b


def _unfold(x, k):
    B, C, H, W = x.shape
    Hp, Wp = H // k, W // k
    u = x.reshape(B, C, Hp, k, Wp, k)
    return jnp.transpose(u, (0, 2, 4, 3, 5, 1)).reshape(B, Hp * Wp, k * k, C)


def _select_body(K, L, v, C, u_ref, o_ref):
    u = u_ref[0]
    ub = u.astype(jnp.bfloat16).astype(jnp.float32)
    s3 = jnp.sum(ub, axis=1, keepdims=True)
    rel = jnp.sum(ub * s3, axis=2, keepdims=True) * (1.0 / v)
    ii = jax.lax.broadcasted_iota(jnp.int32, (L, v, 1), 1)
    for kk in range(K):
        m = jnp.min(rel, axis=1, keepdims=True)
        ismin = rel == m
        idx = jnp.min(jnp.where(ismin, ii, v), axis=1, keepdims=True)
        mask = ii == idx
        o_ref[0, :, kk, :] = jnp.sum(u * mask.astype(jnp.float32), axis=1)
        rel = jnp.where(mask, jnp.inf, rel)


def _select(u, K):
    B, L, v, C = u.shape
    out = pl.pallas_call(
        functools.partial(_select_body, K, L, v, C),
        grid=(B,),
        in_specs=[pl.BlockSpec((1, L, v, C), lambda b: (b, 0, 0, 0))],
        out_specs=pl.BlockSpec((1, L, K, C), lambda b: (b, 0, 0, 0)),
        out_shape=jax.ShapeDtypeStruct((B, L, K, C), jnp.float32),
    )(u)
    return out.reshape(B, L * K, C)


def _rel3_body(shapes, u0, u1, u2, o0, o1, o2):
    for u_ref, o_ref, (L, v, vp, C) in zip((u0, u1, u2), (o0, o1, o2), shapes):
        u = u_ref[0]
        ub = u.astype(jnp.bfloat16).astype(jnp.float32)
        s3 = jnp.sum(ub, axis=1, keepdims=True)
        rel = jnp.sum(ub * s3, axis=2, keepdims=True) * (1.0 / v)
        o_ref[0, :, 0:v, :] = rel
        if vp > v:
            o_ref[0, :, v:vp, :] = jnp.full((L, vp - v, 1), jnp.inf, jnp.float32)


def _rel3(us, vps):
    B = us[0].shape[0]
    shapes = tuple((u.shape[1], u.shape[2], vp, u.shape[3])
                   for u, vp in zip(us, vps))
    outs = pl.pallas_call(
        functools.partial(_rel3_body, shapes),
        grid=(B,),
        in_specs=[pl.BlockSpec((1, L, v, C), lambda b: (b, 0, 0, 0))
                  for (L, v, vp, C) in shapes],
        out_specs=[pl.BlockSpec((1, L, vp, 1), lambda b: (b, 0, 0, 0))
                   for (L, v, vp, C) in shapes],
        out_shape=[jax.ShapeDtypeStruct((B, L, vp, 1), jnp.float32)
                   for (L, v, vp, C) in shapes],
    )(*us)
    return [o.reshape(-1, vp) for o, (L, v, vp, C) in zip(outs, shapes)]


_NC = 2
_NS = 16
_NW = _NC * _NS
_PPW = 8
_IOTA16 = None


def _lane_perm(x, perm):
    dn = lax.GatherDimensionNumbers(offset_dims=(), collapsed_slice_dims=(0,),
                                    start_index_map=(0,))
    return lax.gather(x, perm[:, None], dn, (1,),
                      mode=lax.GatherScatterMode.PROMISE_IN_BOUNDS)


def _lane_min_bcast(x, ii):
    for sh in (8, 4, 2, 1):
        x = jnp.minimum(x, _lane_perm(x, jnp.bitwise_and(ii + sh, 15)))
    return x

def _sc_select_body(cfg, *refs):
    nlvl = len(cfg)
    rel_refs = refs[0:nlvl]
    u_refs = refs[nlvl:2 * nlvl]
    o_refs = refs[2 * nlvl:3 * nlvl]
    scr = refs[3 * nlvl:]
    sem = scr[-1]
    wid = lax.axis_index("s") * _NC + lax.axis_index("c")
    ii = lax.broadcasted_iota(jnp.int32, (16,), 0)
    big = jnp.int32(1 << 20)
    inf = jnp.float32(jnp.inf)
    base = wid * _PPW
    basev = jnp.full((16,), wid, jnp.int32) * _PPW
    rsems = scr[-2 * nlvl - 1:-nlvl - 1]
    gsems = scr[-nlvl - 1:-1]
    rcps = []
    for lvl in range(nlvl):
        P, v, vp, K, C, _Creal = cfg[lvl]
        rv = scr[3 * lvl + 0]
        rcps.append(pltpu.async_copy(
            rel_refs[lvl].at[pl.ds(base * vp, _PPW * vp)], rv, rsems[lvl]))
    gcps = []
    for lvl in range(nlvl):
        P, v, vp, K, C, _Creal = cfg[lvl]
        rv = scr[3 * lvl + 0]
        ib = scr[3 * lvl + 1]
        rw = scr[3 * lvl + 2]
        rcps[lvl].wait()
        nj = vp // 16
        nib = -(-(_PPW * K) // 16)
        acc = [jnp.zeros((16,), jnp.int32) for _ in range(nib)]
        for p in range(_PPW):
            cur = [rv[pl.ds(p * vp + j * 16, 16)] for j in range(nj)]
            for kk in range(K):
                m = _lane_min_bcast(functools.reduce(jnp.minimum, cur), ii)
                cand = [jnp.where(cur[j] == m, ii + j * 16, big)
                        for j in range(nj)]
                idxv = _lane_min_bcast(functools.reduce(jnp.minimum, cand), ii)
                for j in range(nj):
                    cur[j] = jnp.where(ii + j * 16 == idxv, inf, cur[j])
                g = jnp.minimum((basev + p) * v + idxv, P * v - 1)
                pos = p * K + kk
                acc[pos // 16] = jnp.where(ii == (pos % 16), g, acc[pos // 16])
        for j in range(nib):
            ib[pl.ds(j * 16, 16)] = acc[j]
        gcps.append(pltpu.async_copy(u_refs[lvl].at[ib], rw, gsems[lvl]))
    for lvl in range(nlvl):
        P, v, vp, K, C, _Creal = cfg[lvl]
        rw = scr[3 * lvl + 2]
        gcps[lvl].wait()
        pltpu.sync_copy(rw.at[pl.ds(0, _PPW * K)],
                        o_refs[lvl].at[pl.ds(base * K, _PPW * K)])


def _sc_select(rels, us, cfg):
    mesh = plsc.VectorSubcoreMesh(core_axis_name="c", subcore_axis_name="s")
    out_type = [jax.ShapeDtypeStruct((_NW * _PPW * K, C), jnp.float32)
                for (P, v, vp, K, C, _Cr) in cfg]
    scratch = []
    for (P, v, vp, K, C, _Cr) in cfg:
        nib = -(-(_PPW * K) // 16) * 16
        scratch += [pltpu.VMEM((_PPW * vp,), jnp.float32),
                    pltpu.VMEM((nib,), jnp.int32),
                    pltpu.VMEM((nib, C), jnp.float32)]
    for _ in range(2 * len(cfg)):
        scratch.append(pltpu.SemaphoreType.DMA)
    scratch.append(pltpu.SemaphoreType.DMA)
    fn = pl.kernel(functools.partial(_sc_select_body, cfg),
                   out_type=out_type, mesh=mesh, scratch_types=scratch)
    return fn(*rels, *us)


def _attn_body(Ci, Ms, Nb, q_ref,
               s0, s1, s2, qg, qb, kg0, kb0, kg1, kb1, kg2, kb2,
               wq, wk0, wv0, wk1, wv1, wk2, wv2, wout,
               o_ref, k_scr, v_scr, os_scr):
    dh = Ci
    sk = max(dh, 128)
    s_refs = (s0, s1, s2)
    kg = (kg0, kg1, kg2)
    kb = (kb0, kb1, kb2)
    wk = (wk0, wk1, wk2)
    wv = (wv0, wv1, wv2)
    offs = (0, Ms[0], Ms[0] + Ms[1])

    @pl.when(pl.program_id(1) == 0)
    def _():
        for c in range(3):
            ln = _ln(s_refs[c][0], kg[c][0], kb[c][0])
            k_scr[offs[c]:offs[c] + Ms[c], :] = jax.lax.dot(ln, wk[c][...], precision=_PREC)
            v_scr[offs[c]:offs[c] + Ms[c], :] = jax.lax.dot(ln, wv[c][...], precision=_PREC)

    x = q_ref[0]
    mu = jnp.mean(x, axis=0, keepdims=True)
    var = jnp.mean((x - mu) ** 2, axis=0, keepdims=True)
    x_ln = (x - mu) / jnp.sqrt(var + 1e-5) * qg[...] + qb[...]
    q_all = jax.lax.dot_general(x_ln, wq[...], (((0,), (0,)), ((), ())),
                                precision=_PREC)
    for h in range(_H):
        qh = q_all[:, h * sk:(h + 1) * sk]
        for c in range(3):
            kc = k_scr[offs[c]:offs[c] + Ms[c], h * sk:(h + 1) * sk]
            dots = jax.lax.dot_general(qh, kc, (((1,), (1,)), ((), ())), precision=_PREC)
            e = jnp.exp(dots)
            r = 1.0 / jnp.sum(e, axis=1, keepdims=True)
            vc = v_scr[offs[c]:offs[c] + Ms[c], h * sk:(h + 1) * sk]
            av = jax.lax.dot(e, vc, precision=_PREC)
            os_scr[:, (h * 3 + c) * sk:(h * 3 + c + 1) * sk] = av * r
    o_ref[0] = jax.lax.dot_general(wout[...], os_scr[...], (((0,), (1,)), ((), ())),
                                   precision=_PREC)


def _pad_heads(w, dh, sk):
    if sk == dh:
        return w
    rows = w.shape[0]
    w = w.reshape(rows, _H, dh)
    w = jnp.pad(w, ((0, 0), (0, 0), (0, sk - dh)))
    return w.reshape(rows, _H * sk)


def _attn(query, skips, ap, i):
    B, Ci, N = query.shape
    dh = Ci
    sk = max(dh, 128)
    inner = _H * Ci
    Ms = tuple(s.shape[1] for s in skips)
    ds = tuple(s.shape[2] for s in skips)
    Mtot = sum(Ms)
    Nb = _NB[i]
    nblk = pl.cdiv(N, Nb)
    scale = dh ** -0.5

    wq = _pad_heads(ap['Wq'].T * scale, dh, sk)
    wks, wvs = [], []
    for c in range(3):
        wkvT = ap['Wkv'][c].T
        wks.append(_pad_heads(wkvT[:, :inner], dh, sk))
        wvs.append(_pad_heads(wkvT[:, inner:], dh, sk))
    woutT = ap['Wout'].T
    if sk != dh:
        w3 = woutT.reshape(3 * _H, dh, Ci)
        w3 = jnp.pad(w3, ((0, 0), (0, sk - dh), (0, 0)))
        woutT = w3.reshape(3 * _H * sk, Ci)

    def full(a):
        nd = a.ndim
        return pl.BlockSpec(a.shape, lambda b, n: (0,) * nd)

    qg = ap['qn_g'].reshape(Ci, 1)
    qb = ap['qn_b'].reshape(Ci, 1)
    kgs = [g.reshape(1, -1) for g in ap['kvn_g']]
    kbs = [b.reshape(1, -1) for b in ap['kvn_b']]

    operands = [query] + list(skips) + [qg, qb,
                kgs[0], kbs[0], kgs[1], kbs[1], kgs[2], kbs[2],
                wq, wks[0], wvs[0], wks[1], wvs[1], wks[2], wvs[2], woutT]
    in_specs = [pl.BlockSpec((1, Ci, Nb), lambda b, n: (b, 0, n))]
    for c in range(3):
        in_specs.append(pl.BlockSpec((1, Ms[c], ds[c]),
                                     lambda b, n: (b, 0, 0)))
    for a in operands[4:]:
        in_specs.append(full(a))

    return pl.pallas_call(
        functools.partial(_attn_body, Ci, Ms, Nb),
        grid=(B, nblk),
        in_specs=in_specs,
        out_specs=pl.BlockSpec((1, Ci, Nb), lambda b, n: (b, 0, n)),
        out_shape=jax.ShapeDtypeStruct((B, Ci, N), jnp.float32),
        scratch_shapes=[
            pltpu.VMEM((Mtot, _H * sk), jnp.float32),
            pltpu.VMEM((Mtot, _H * sk), jnp.float32),
            pltpu.VMEM((Nb, 3 * _H * sk), jnp.float32),
        ],
    )(*operands)


def _smla(xs, blocks):
    B = xs[0].shape[0]
    tmp_q, tmp_sk = [], []
    rels, uflats, cfg, us, vps = [], [], [], [], []
    for i, x in enumerate(xs):
        C, Hh, Ww = _SHAPES[i]
        tmp_q.append(x.reshape(B, C, Hh * Ww))
        u = _unfold(x, _KS[i])
        if _KS[i] == 1:
            tmp_sk.append(u.reshape(B, -1, C))
            continue
        L, v = u.shape[1], u.shape[2]
        vp = max(16, -(-v // 16) * 16)
        P = B * L
        ppad = _NW * _PPW
        us.append(u)
        vps.append(vp)
        uf = u.reshape(P * v, C)
        if C < 128:
            uf = jnp.pad(uf, ((0, 0), (0, 128 - C)))
        uflats.append(uf)
        cfg.append((P, v, vp, _TOPK[i], max(C, 128), C))
        tmp_sk.append(i)
    ppad = _NW * _PPW
    for r, (P, v, vp, K, Cp, C) in zip(_rel3(us, vps), cfg):
        r = jnp.pad(r, ((0, ppad - P), (0, 0)), constant_values=jnp.inf)
        rels.append(r.reshape(-1))
    sels = _sc_select(rels, uflats, tuple(cfg))
    si = 0
    for i in range(4):
        if isinstance(tmp_sk[i], int):
            P, v, vp, K, Cp, C = cfg[si]
            L = P // B
            tmp_sk[i] = sels[si][:P * K, :C].reshape(B, L * K, C)
            si += 1
    new = []
    for idx in range(4):
        new.append(_attn(tmp_q[idx],
                         [tmp_sk[j] for j in range(4) if j != idx],
                         blocks[idx], idx))
    outs = []
    for i, ns in enumerate(new):
        C, Hh, Ww = _SHAPES[i]
        outs.append(ns.reshape(B, C, Hh, Ww))
    return outs


def kernel(x0, x1, x2, x3, params):
    xs = [x0, x1, x2, x3]
    for t in range(_TNUM):
        xs = _smla(xs, params[t])
    return tuple(xs)
